# baseline (speedup 1.0000x reference)
import functools

import jax
import jax.numpy as jnp
from jax import lax
from jax.experimental import pallas as pl
from jax.experimental.pallas import tpu as pltpu
from jax.experimental.pallas import tpu_sc as plsc

N = 10000
E = 320000
D = 128

NC = 2
NS = 16
NW = NC * NS

PAIRS = 2 * E
PW = 20480
PPAD = NW * PW
CAP = 12800
CHG = 64
CPR = 128 // CHG
NBUF = 4
PAD_CH = NBUF * CHG
HALF = 5000
NRA = 5008
TRASH = HALF
RPT = 320
RPT_LAST = NRA - 15 * RPT

BM = 1000


def _sc_prepass_body(gsrc_hbm, tgt_hbm, lg0_hbm, ls0_hbm, lg1_hbm, ls1_hbm,
                     cnts_hbm, g_in, t_in, lg0_v, ls0_v, lg1_v, ls1_v, cnt_v):
    c = lax.axis_index("c")
    s = lax.axis_index("s")
    w = c * NS + s

    pltpu.sync_copy(gsrc_hbm.at[w], g_in)
    pltpu.sync_copy(tgt_hbm.at[w], t_in)

    def step(k, carry):
        c0, c1 = carry
        row = k // 8
        col = (k % 8) * 16
        t = t_in[row, pl.ds(col, 16)]
        g = g_in[row, pl.ds(col, 16)]
        m0 = (t >= 0) & (t < HALF)
        m1 = t >= HALF
        i0 = m0.astype(jnp.int32)
        i1 = m1.astype(jnp.int32)
        r0 = plsc.cumsum(i0)
        r1 = plsc.cumsum(i1)
        pos0 = jnp.where(m0, c0 + r0 - 1, CAP - 1)
        pos1 = jnp.where(m1, c1 + r1 - 1, CAP - 1)
        plsc.store_scatter(lg0_v, [pos0], g)
        plsc.store_scatter(ls0_v, [pos0], t)
        plsc.store_scatter(lg1_v, [pos1], g)
        plsc.store_scatter(ls1_v, [pos1], t - HALF)
        c0 = jnp.minimum(c0 + jnp.sum(i0), CAP - 2 * PAD_CH)
        c1 = jnp.minimum(c1 + jnp.sum(i1), CAP - 2 * PAD_CH)
        return c0, c1

    c0, c1 = lax.fori_loop(0, PW // 16, step, (jnp.int32(0), jnp.int32(0)),
                           unroll=False)

    zero16 = jnp.zeros((16,), jnp.int32)
    trash16 = zero16 + TRASH
    for r in range(PAD_CH // 16):
        lg0_v[pl.ds(c0 + r * 16, 16)] = zero16
        ls0_v[pl.ds(c0 + r * 16, 16)] = trash16
        lg1_v[pl.ds(c1 + r * 16, 16)] = zero16
        ls1_v[pl.ds(c1 + r * 16, 16)] = trash16
    c0p = ((c0 + PAD_CH - 1) // PAD_CH) * PAD_CH
    c1p = ((c1 + PAD_CH - 1) // PAD_CH) * PAD_CH

    io = lax.iota(jnp.int32, 16)
    cnt_v[...] = jnp.where(io == 0, c0p, jnp.where(io == 1, c1p, 0))

    pltpu.sync_copy(lg0_v, lg0_hbm.at[pl.ds(w * CAP, CAP)])
    pltpu.sync_copy(ls0_v, ls0_hbm.at[pl.ds(w * CAP, CAP)])
    pltpu.sync_copy(lg1_v, lg1_hbm.at[pl.ds(w * CAP, CAP)])
    pltpu.sync_copy(ls1_v, ls1_hbm.at[pl.ds(w * CAP, CAP)])
    pltpu.sync_copy(cnt_v, cnts_hbm.at[pl.ds(w * 16, 16)])


@functools.partial(
    pl.kernel,
    mesh=plsc.VectorSubcoreMesh(core_axis_name="c", subcore_axis_name="s"),
    compiler_params=pltpu.CompilerParams(needs_layout_passes=False),
    out_type=[
        jax.ShapeDtypeStruct((NW * CAP,), jnp.int32),
        jax.ShapeDtypeStruct((NW * CAP,), jnp.int32),
        jax.ShapeDtypeStruct((NW * CAP,), jnp.int32),
        jax.ShapeDtypeStruct((NW * CAP,), jnp.int32),
        jax.ShapeDtypeStruct((NW * 16,), jnp.int32),
    ],
    scratch_types=[
        pltpu.VMEM((PW // 128, 128), jnp.int32),
        pltpu.VMEM((PW // 128, 128), jnp.int32),
        pltpu.VMEM((CAP,), jnp.int32),
        pltpu.VMEM((CAP,), jnp.int32),
        pltpu.VMEM((CAP,), jnp.int32),
        pltpu.VMEM((CAP,), jnp.int32),
        pltpu.VMEM((16,), jnp.int32),
    ],
)
def _sc_prepass(*refs):
    _sc_prepass_body(*refs)


def _sc_scatter_body(table_hbm, lg0_hbm, lg1_hbm, ls0_hbm, ls1_hbm, cnts_hbm,
                     zeros_hbm, out_hbm,
                     lg_v, ls_v, cnt_v, rows_a, rows_b, rows_c, rows_d,
                     acc_sh, sem_a, sem_b, sem_c, sem_d):
    c = lax.axis_index("c")
    s = lax.axis_index("s")

    @pl.when(s < NS - 1)
    def _():
        pltpu.sync_copy(zeros_hbm.at[pl.ds(s * RPT, RPT)],
                        acc_sh.at[pl.ds(s * RPT, RPT)])

    @pl.when(s == NS - 1)
    def _():
        pltpu.sync_copy(zeros_hbm.at[pl.ds((NS - 1) * RPT, RPT_LAST)],
                        acc_sh.at[pl.ds((NS - 1) * RPT, RPT_LAST)])

    plsc.subcore_barrier()

    bufs = [(rows_a, sem_a), (rows_b, sem_b), (rows_c, sem_c), (rows_d, sem_d)]

    def gather(k, buf, sem):
        pltpu.async_copy(table_hbm.at[lg_v.at[pl.ds(k * CHG, CHG)]], buf, sem)

    def drain_scatter(k, buf, sem):
        pltpu.make_async_copy(
            table_hbm.at[lg_v.at[pl.ds(k * CHG, CHG)]], buf, sem).wait()
        pltpu.sync_copy(
            buf, acc_sh.at[ls_v.at[k // CPR, pl.ds((k % CPR) * CHG, CHG)]],
            add=True)

    for li in range(2):
        w = 2 * s + li

        @pl.when(c == 0)
        def _():
            pltpu.sync_copy(lg0_hbm.at[pl.ds(w * CAP, CAP)], lg_v)
            pltpu.sync_copy(ls0_hbm.at[w], ls_v)

        @pl.when(c == 1)
        def _():
            pltpu.sync_copy(lg1_hbm.at[pl.ds(w * CAP, CAP)], lg_v)
            pltpu.sync_copy(ls1_hbm.at[w], ls_v)

        pltpu.sync_copy(cnts_hbm.at[pl.ds(w * 16, 16)], cnt_v)
        io = lax.iota(jnp.int32, 16)
        cnt = jnp.sum(jnp.where(io == c, cnt_v[...], 0))
        nch = cnt // CHG
        n4 = cnt // PAD_CH

        for q in range(NBUF - 1):
            @pl.when(q < nch)
            def _():
                gather(q, *bufs[q])

        def chunk(j, carry):
            for q in range(NBUF):
                k = NBUF * j + q
                kn = k + NBUF - 1

                @pl.when(kn < nch)
                def _():
                    gather(kn, *bufs[(q + NBUF - 1) % NBUF])

                drain_scatter(k, *bufs[q])
            return carry

        lax.fori_loop(0, n4, chunk, 0, unroll=False)

    plsc.subcore_barrier()

    @pl.when(s < NS - 1)
    def _():
        pltpu.sync_copy(acc_sh.at[pl.ds(s * RPT, RPT)],
                        out_hbm.at[c, pl.ds(s * RPT, RPT)])

    @pl.when(s == NS - 1)
    def _():
        pltpu.sync_copy(acc_sh.at[pl.ds((NS - 1) * RPT, RPT_LAST)],
                        out_hbm.at[c, pl.ds((NS - 1) * RPT, RPT_LAST)])


@functools.partial(
    pl.kernel,
    mesh=plsc.VectorSubcoreMesh(core_axis_name="c", subcore_axis_name="s"),
    compiler_params=pltpu.CompilerParams(needs_layout_passes=False),
    out_type=jax.ShapeDtypeStruct((NC, NRA, D), jnp.float32),
    scratch_types=[
        pltpu.VMEM((CAP,), jnp.int32),
        pltpu.VMEM((CAP // 128, 128), jnp.int32),
        pltpu.VMEM((16,), jnp.int32),
        pltpu.VMEM((CHG, D), jnp.float32),
        pltpu.VMEM((CHG, D), jnp.float32),
        pltpu.VMEM((CHG, D), jnp.float32),
        pltpu.VMEM((CHG, D), jnp.float32),
        pltpu.VMEM_SHARED((NRA, D), jnp.float32),
        pltpu.SemaphoreType.DMA,
        pltpu.SemaphoreType.DMA,
        pltpu.SemaphoreType.DMA,
        pltpu.SemaphoreType.DMA,
    ],
)
def _sc_scatter(*refs):
    _sc_scatter_body(*refs)


def _lin2_kernel(x_ref, w0_ref, b0_ref, w1_ref, b1_ref, o0_ref, o1_ref):
    x = x_ref[...]
    o0_ref[...] = jnp.dot(x, w0_ref[...], preferred_element_type=jnp.float32) + b0_ref[...]
    o1_ref[...] = jnp.dot(x, w1_ref[...], preferred_element_type=jnp.float32) + b1_ref[...]


def _lin2_fused_kernel(vw0_ref, ns_ref, w0_ref, b0_ref, w1_ref, b1_ref,
                       o0_ref, o1_ref):
    h = jax.nn.relu(vw0_ref[...] + ns_ref[0])
    o0_ref[...] = jnp.dot(h, w0_ref[...], preferred_element_type=jnp.float32) + b0_ref[...]
    o1_ref[...] = jnp.dot(h, w1_ref[...], preferred_element_type=jnp.float32) + b1_ref[...]


def _final_kernel(x_ref, vw0_ref, ns_ref, o_ref):
    h = jax.nn.relu(vw0_ref[...] + ns_ref[0])
    o_ref[...] = (x_ref[...] + h) * 0.5


_W_SPEC = pl.BlockSpec((D, D), lambda i: (0, 0))
_B_SPEC = pl.BlockSpec((1, D), lambda i: (0, 0))
_X_SPEC = pl.BlockSpec((BM, D), lambda i: (i, 0))
_NS_SPEC = pl.BlockSpec((1, BM, D), lambda i: (i // 5, i % 5, 0))
_OUT2_SHAPE = [jax.ShapeDtypeStruct((N, D), jnp.float32)] * 2


def _lin2(x, w0t, b0, w1t, b1):
    return pl.pallas_call(
        _lin2_kernel,
        grid=(N // BM,),
        in_specs=[_X_SPEC, _W_SPEC, _B_SPEC, _W_SPEC, _B_SPEC],
        out_specs=[_X_SPEC, _X_SPEC],
        out_shape=_OUT2_SHAPE,
    )(x, w0t, b0.reshape(1, D), w1t, b1.reshape(1, D))


def _lin2_fused(vw0, ns, w0t, b0, w1t, b1):
    return pl.pallas_call(
        _lin2_fused_kernel,
        grid=(N // BM,),
        in_specs=[_X_SPEC, _NS_SPEC, _W_SPEC, _B_SPEC, _W_SPEC, _B_SPEC],
        out_specs=[_X_SPEC, _X_SPEC],
        out_shape=_OUT2_SHAPE,
    )(vw0, ns, w0t, b0.reshape(1, D), w1t, b1.reshape(1, D))


def _final(x, vw0, ns):
    return pl.pallas_call(
        _final_kernel,
        grid=(N // BM,),
        in_specs=[_X_SPEC, _X_SPEC, _NS_SPEC],
        out_specs=_X_SPEC,
        out_shape=jax.ShapeDtypeStruct((N, D), jnp.float32),
    )(x, vw0, ns)


def kernel(input_feats, edges_packed, W0_1, b0_1, W1_1, b1_1,
           W0_2, b0_2, W1_2, b1_2):
    src = edges_packed[:, 0].astype(jnp.int32)
    dst = edges_packed[:, 1].astype(jnp.int32)
    padz = jnp.zeros((PPAD - PAIRS,), jnp.int32)
    gsrc = jnp.concatenate([dst, src, padz]).reshape(NW, PW // 128, 128)
    tgt = jnp.concatenate([src, dst, padz - 1]).reshape(NW, PW // 128, 128)

    lg0, ls0, lg1, ls1, cnts = _sc_prepass(gsrc, tgt)
    ls0 = ls0.reshape(NW, CAP // 128, 128)
    ls1 = ls1.reshape(NW, CAP // 128, 128)
    zeros = jnp.zeros((NRA, D), jnp.float32)

    vw0_1, vw1_1 = _lin2(input_feats, W0_1.T, b0_1, W1_1.T, b1_1)
    ns1 = _sc_scatter(vw1_1, lg0, lg1, ls0, ls1, cnts, zeros)
    vw0_2, vw1_2 = _lin2_fused(vw0_1, ns1, W0_2.T, b0_2, W1_2.T, b1_2)
    ns2 = _sc_scatter(vw1_2, lg0, lg1, ls0, ls1, cnts, zeros)
    return _final(input_feats, vw0_2, ns2)


# speedup vs baseline: 1.3105x; 1.3105x over previous
import functools

import jax
import jax.numpy as jnp
from jax import lax
from jax.experimental import pallas as pl
from jax.experimental.pallas import tpu as pltpu
from jax.experimental.pallas import tpu_sc as plsc

N = 10000
E = 320000
D = 128

NC = 2
NS = 16
NW = NC * NS

PAIRS = 2 * E
PW = 20480
PPAD = NW * PW
CAP = 12800
CHG = 16
CPR = 128 // CHG
NBUF = 8
PAD_CH = NBUF * CHG
HALF = 5000
NRA = 5008
TRASH = HALF
RPT = 320
RPT_LAST = NRA - 15 * RPT

BM = 1000


def _sc_prepass_body(gsrc_hbm, tgt_hbm, lg0_hbm, ls0_hbm, lg1_hbm, ls1_hbm,
                     cnts_hbm, g_in, t_in, lg0_v, ls0_v, lg1_v, ls1_v, cnt_v):
    c = lax.axis_index("c")
    s = lax.axis_index("s")
    w = c * NS + s

    pltpu.sync_copy(gsrc_hbm.at[w], g_in)
    pltpu.sync_copy(tgt_hbm.at[w], t_in)

    def step(k, carry):
        c0, c1 = carry
        row = k // 8
        col = (k % 8) * 16
        t = t_in[row, pl.ds(col, 16)]
        g = g_in[row, pl.ds(col, 16)]
        m0 = (t >= 0) & (t < HALF)
        m1 = t >= HALF
        i0 = m0.astype(jnp.int32)
        i1 = m1.astype(jnp.int32)
        r0 = plsc.cumsum(i0)
        r1 = plsc.cumsum(i1)
        pos0 = jnp.where(m0, c0 + r0 - 1, CAP - 1)
        pos1 = jnp.where(m1, c1 + r1 - 1, CAP - 1)
        plsc.store_scatter(lg0_v, [pos0], g)
        plsc.store_scatter(ls0_v, [pos0], t)
        plsc.store_scatter(lg1_v, [pos1], g)
        plsc.store_scatter(ls1_v, [pos1], t - HALF)
        c0 = jnp.minimum(c0 + jnp.sum(i0), CAP - 2 * PAD_CH)
        c1 = jnp.minimum(c1 + jnp.sum(i1), CAP - 2 * PAD_CH)
        return c0, c1

    c0, c1 = lax.fori_loop(0, PW // 16, step, (jnp.int32(0), jnp.int32(0)),
                           unroll=False)

    zero16 = jnp.zeros((16,), jnp.int32)
    trash16 = zero16 + TRASH
    for r in range(PAD_CH // 16):
        lg0_v[pl.ds(c0 + r * 16, 16)] = zero16
        ls0_v[pl.ds(c0 + r * 16, 16)] = trash16
        lg1_v[pl.ds(c1 + r * 16, 16)] = zero16
        ls1_v[pl.ds(c1 + r * 16, 16)] = trash16
    c0p = ((c0 + PAD_CH - 1) // PAD_CH) * PAD_CH
    c1p = ((c1 + PAD_CH - 1) // PAD_CH) * PAD_CH

    io = lax.iota(jnp.int32, 16)
    cnt_v[...] = jnp.where(io == 0, c0p, jnp.where(io == 1, c1p, 0))

    pltpu.sync_copy(lg0_v, lg0_hbm.at[pl.ds(w * CAP, CAP)])
    pltpu.sync_copy(ls0_v, ls0_hbm.at[pl.ds(w * CAP, CAP)])
    pltpu.sync_copy(lg1_v, lg1_hbm.at[pl.ds(w * CAP, CAP)])
    pltpu.sync_copy(ls1_v, ls1_hbm.at[pl.ds(w * CAP, CAP)])
    pltpu.sync_copy(cnt_v, cnts_hbm.at[pl.ds(w * 16, 16)])


@functools.partial(
    pl.kernel,
    mesh=plsc.VectorSubcoreMesh(core_axis_name="c", subcore_axis_name="s"),
    compiler_params=pltpu.CompilerParams(needs_layout_passes=False),
    out_type=[
        jax.ShapeDtypeStruct((NW * CAP,), jnp.int32),
        jax.ShapeDtypeStruct((NW * CAP,), jnp.int32),
        jax.ShapeDtypeStruct((NW * CAP,), jnp.int32),
        jax.ShapeDtypeStruct((NW * CAP,), jnp.int32),
        jax.ShapeDtypeStruct((NW * 16,), jnp.int32),
    ],
    scratch_types=[
        pltpu.VMEM((PW // 128, 128), jnp.int32),
        pltpu.VMEM((PW // 128, 128), jnp.int32),
        pltpu.VMEM((CAP,), jnp.int32),
        pltpu.VMEM((CAP,), jnp.int32),
        pltpu.VMEM((CAP,), jnp.int32),
        pltpu.VMEM((CAP,), jnp.int32),
        pltpu.VMEM((16,), jnp.int32),
    ],
)
def _sc_prepass(*refs):
    _sc_prepass_body(*refs)


def _sc_scatter_body(table_hbm, lg0_hbm, lg1_hbm, ls0_hbm, ls1_hbm, cnts_hbm,
                     zeros_hbm, out_hbm,
                     lg_v, ls_v, cnt_v, rows_a, rows_b, rows_c, rows_d,
                     rows_e, rows_f, rows_g, rows_h,
                     acc_sh, sem_a, sem_b, sem_c, sem_d,
                     sem_e, sem_f, sem_g, sem_h):
    c = lax.axis_index("c")
    s = lax.axis_index("s")

    @pl.when(s < NS - 1)
    def _():
        pltpu.sync_copy(zeros_hbm.at[pl.ds(s * RPT, RPT)],
                        acc_sh.at[pl.ds(s * RPT, RPT)])

    @pl.when(s == NS - 1)
    def _():
        pltpu.sync_copy(zeros_hbm.at[pl.ds((NS - 1) * RPT, RPT_LAST)],
                        acc_sh.at[pl.ds((NS - 1) * RPT, RPT_LAST)])

    plsc.subcore_barrier()

    bufs = [(rows_a, sem_a), (rows_b, sem_b), (rows_c, sem_c), (rows_d, sem_d),
            (rows_e, sem_e), (rows_f, sem_f), (rows_g, sem_g), (rows_h, sem_h)]

    def gather(k, buf, sem):
        pltpu.async_copy(table_hbm.at[lg_v.at[pl.ds(k * CHG, CHG)]], buf, sem)

    def drain_scatter(k, buf, sem):
        pltpu.make_async_copy(
            table_hbm.at[lg_v.at[pl.ds(k * CHG, CHG)]], buf, sem).wait()
        pltpu.sync_copy(
            buf, acc_sh.at[ls_v.at[k // CPR, pl.ds((k % CPR) * CHG, CHG)]],
            add=True)

    for li in range(2):
        w = 2 * s + li

        @pl.when(c == 0)
        def _():
            pltpu.sync_copy(lg0_hbm.at[pl.ds(w * CAP, CAP)], lg_v)
            pltpu.sync_copy(ls0_hbm.at[w], ls_v)

        @pl.when(c == 1)
        def _():
            pltpu.sync_copy(lg1_hbm.at[pl.ds(w * CAP, CAP)], lg_v)
            pltpu.sync_copy(ls1_hbm.at[w], ls_v)

        pltpu.sync_copy(cnts_hbm.at[pl.ds(w * 16, 16)], cnt_v)
        io = lax.iota(jnp.int32, 16)
        cnt = jnp.sum(jnp.where(io == c, cnt_v[...], 0))
        nch = cnt // CHG
        n4 = cnt // PAD_CH

        for q in range(NBUF - 1):
            @pl.when(q < nch)
            def _():
                gather(q, *bufs[q])

        def chunk(j, carry):
            for q in range(NBUF):
                k = NBUF * j + q
                kn = k + NBUF - 1

                @pl.when(kn < nch)
                def _():
                    gather(kn, *bufs[(q + NBUF - 1) % NBUF])

                drain_scatter(k, *bufs[q])
            return carry

        lax.fori_loop(0, n4, chunk, 0, unroll=False)

    plsc.subcore_barrier()

    @pl.when(s < NS - 1)
    def _():
        pltpu.sync_copy(acc_sh.at[pl.ds(s * RPT, RPT)],
                        out_hbm.at[c, pl.ds(s * RPT, RPT)])

    @pl.when(s == NS - 1)
    def _():
        pltpu.sync_copy(acc_sh.at[pl.ds((NS - 1) * RPT, RPT_LAST)],
                        out_hbm.at[c, pl.ds((NS - 1) * RPT, RPT_LAST)])


@functools.partial(
    pl.kernel,
    mesh=plsc.VectorSubcoreMesh(core_axis_name="c", subcore_axis_name="s"),
    compiler_params=pltpu.CompilerParams(needs_layout_passes=False),
    out_type=jax.ShapeDtypeStruct((NC, NRA, D), jnp.float32),
    scratch_types=[
        pltpu.VMEM((CAP,), jnp.int32),
        pltpu.VMEM((CAP // 128, 128), jnp.int32),
        pltpu.VMEM((16,), jnp.int32),
        pltpu.VMEM((CHG, D), jnp.float32),
        pltpu.VMEM((CHG, D), jnp.float32),
        pltpu.VMEM((CHG, D), jnp.float32),
        pltpu.VMEM((CHG, D), jnp.float32),
        pltpu.VMEM((CHG, D), jnp.float32),
        pltpu.VMEM((CHG, D), jnp.float32),
        pltpu.VMEM((CHG, D), jnp.float32),
        pltpu.VMEM((CHG, D), jnp.float32),
        pltpu.VMEM_SHARED((NRA, D), jnp.float32),
        pltpu.SemaphoreType.DMA,
        pltpu.SemaphoreType.DMA,
        pltpu.SemaphoreType.DMA,
        pltpu.SemaphoreType.DMA,
        pltpu.SemaphoreType.DMA,
        pltpu.SemaphoreType.DMA,
        pltpu.SemaphoreType.DMA,
        pltpu.SemaphoreType.DMA,
    ],
)
def _sc_scatter(*refs):
    _sc_scatter_body(*refs)


def _lin2_kernel(x_ref, w0_ref, b0_ref, w1_ref, b1_ref, o0_ref, o1_ref):
    x = x_ref[...]
    o0_ref[...] = jnp.dot(x, w0_ref[...], preferred_element_type=jnp.float32) + b0_ref[...]
    o1_ref[...] = jnp.dot(x, w1_ref[...], preferred_element_type=jnp.float32) + b1_ref[...]


def _lin2_fused_kernel(vw0_ref, ns_ref, w0_ref, b0_ref, w1_ref, b1_ref,
                       o0_ref, o1_ref):
    h = jax.nn.relu(vw0_ref[...] + ns_ref[0])
    o0_ref[...] = jnp.dot(h, w0_ref[...], preferred_element_type=jnp.float32) + b0_ref[...]
    o1_ref[...] = jnp.dot(h, w1_ref[...], preferred_element_type=jnp.float32) + b1_ref[...]


def _final_kernel(x_ref, vw0_ref, ns_ref, o_ref):
    h = jax.nn.relu(vw0_ref[...] + ns_ref[0])
    o_ref[...] = (x_ref[...] + h) * 0.5


_W_SPEC = pl.BlockSpec((D, D), lambda i: (0, 0))
_B_SPEC = pl.BlockSpec((1, D), lambda i: (0, 0))
_X_SPEC = pl.BlockSpec((BM, D), lambda i: (i, 0))
_NS_SPEC = pl.BlockSpec((1, BM, D), lambda i: (i // 5, i % 5, 0))
_OUT2_SHAPE = [jax.ShapeDtypeStruct((N, D), jnp.float32)] * 2


def _lin2(x, w0t, b0, w1t, b1):
    return pl.pallas_call(
        _lin2_kernel,
        grid=(N // BM,),
        in_specs=[_X_SPEC, _W_SPEC, _B_SPEC, _W_SPEC, _B_SPEC],
        out_specs=[_X_SPEC, _X_SPEC],
        out_shape=_OUT2_SHAPE,
    )(x, w0t, b0.reshape(1, D), w1t, b1.reshape(1, D))


def _lin2_fused(vw0, ns, w0t, b0, w1t, b1):
    return pl.pallas_call(
        _lin2_fused_kernel,
        grid=(N // BM,),
        in_specs=[_X_SPEC, _NS_SPEC, _W_SPEC, _B_SPEC, _W_SPEC, _B_SPEC],
        out_specs=[_X_SPEC, _X_SPEC],
        out_shape=_OUT2_SHAPE,
    )(vw0, ns, w0t, b0.reshape(1, D), w1t, b1.reshape(1, D))


def _final(x, vw0, ns):
    return pl.pallas_call(
        _final_kernel,
        grid=(N // BM,),
        in_specs=[_X_SPEC, _X_SPEC, _NS_SPEC],
        out_specs=_X_SPEC,
        out_shape=jax.ShapeDtypeStruct((N, D), jnp.float32),
    )(x, vw0, ns)


def kernel(input_feats, edges_packed, W0_1, b0_1, W1_1, b1_1,
           W0_2, b0_2, W1_2, b1_2):
    src = edges_packed[:, 0].astype(jnp.int32)
    dst = edges_packed[:, 1].astype(jnp.int32)
    padz = jnp.zeros((PPAD - PAIRS,), jnp.int32)
    gsrc = jnp.concatenate([dst, src, padz]).reshape(NW, PW // 128, 128)
    tgt = jnp.concatenate([src, dst, padz - 1]).reshape(NW, PW // 128, 128)

    lg0, ls0, lg1, ls1, cnts = _sc_prepass(gsrc, tgt)
    ls0 = ls0.reshape(NW, CAP // 128, 128)
    ls1 = ls1.reshape(NW, CAP // 128, 128)
    zeros = jnp.zeros((NRA, D), jnp.float32)

    vw0_1, vw1_1 = _lin2(input_feats, W0_1.T, b0_1, W1_1.T, b1_1)
    ns1 = _sc_scatter(vw1_1, lg0, lg1, ls0, ls1, cnts, zeros)
    vw0_2, vw1_2 = _lin2_fused(vw0_1, ns1, W0_2.T, b0_2, W1_2.T, b1_2)
    ns2 = _sc_scatter(vw1_2, lg0, lg1, ls0, ls1, cnts, zeros)
    return _final(input_feats, vw0_2, ns2)


# speedup vs baseline: 1.3245x; 1.0107x over previous
import functools

import jax
import jax.numpy as jnp
from jax import lax
from jax.experimental import pallas as pl
from jax.experimental.pallas import tpu as pltpu
from jax.experimental.pallas import tpu_sc as plsc

N = 10000
E = 320000
D = 128

NC = 2
NS = 16
NW = NC * NS

PAIRS = 2 * E
PW = 20480
PPAD = NW * PW
CAP = 12800
CHG = 32
CPR = 128 // CHG
NBUF = 4
PAD_CH = NBUF * CHG
HALF = 5000
NRA = 5008
TRASH = HALF
RPT = 320
RPT_LAST = NRA - 15 * RPT

BM = 1000


def _sc_prepass_body(gsrc_hbm, tgt_hbm, lg0_hbm, ls0_hbm, lg1_hbm, ls1_hbm,
                     cnts_hbm, g_in, t_in, lg0_v, ls0_v, lg1_v, ls1_v, cnt_v):
    c = lax.axis_index("c")
    s = lax.axis_index("s")
    w = c * NS + s

    pltpu.sync_copy(gsrc_hbm.at[w], g_in)
    pltpu.sync_copy(tgt_hbm.at[w], t_in)

    def step(k, carry):
        c0, c1 = carry
        row = k // 8
        col = (k % 8) * 16
        t = t_in[row, pl.ds(col, 16)]
        g = g_in[row, pl.ds(col, 16)]
        m0 = (t >= 0) & (t < HALF)
        m1 = t >= HALF
        i0 = m0.astype(jnp.int32)
        i1 = m1.astype(jnp.int32)
        r0 = plsc.cumsum(i0)
        r1 = plsc.cumsum(i1)
        pos0 = jnp.where(m0, c0 + r0 - 1, CAP - 1)
        pos1 = jnp.where(m1, c1 + r1 - 1, CAP - 1)
        plsc.store_scatter(lg0_v, [pos0], g)
        plsc.store_scatter(ls0_v, [pos0], t)
        plsc.store_scatter(lg1_v, [pos1], g)
        plsc.store_scatter(ls1_v, [pos1], t - HALF)
        c0 = jnp.minimum(c0 + jnp.sum(i0), CAP - 2 * PAD_CH)
        c1 = jnp.minimum(c1 + jnp.sum(i1), CAP - 2 * PAD_CH)
        return c0, c1

    c0, c1 = lax.fori_loop(0, PW // 16, step, (jnp.int32(0), jnp.int32(0)),
                           unroll=False)

    zero16 = jnp.zeros((16,), jnp.int32)
    trash16 = zero16 + TRASH
    for r in range(PAD_CH // 16):
        lg0_v[pl.ds(c0 + r * 16, 16)] = zero16
        ls0_v[pl.ds(c0 + r * 16, 16)] = trash16
        lg1_v[pl.ds(c1 + r * 16, 16)] = zero16
        ls1_v[pl.ds(c1 + r * 16, 16)] = trash16
    c0p = ((c0 + PAD_CH - 1) // PAD_CH) * PAD_CH
    c1p = ((c1 + PAD_CH - 1) // PAD_CH) * PAD_CH

    io = lax.iota(jnp.int32, 16)
    cnt_v[...] = jnp.where(io == 0, c0p, jnp.where(io == 1, c1p, 0))

    pltpu.sync_copy(lg0_v, lg0_hbm.at[pl.ds(w * CAP, CAP)])
    pltpu.sync_copy(ls0_v, ls0_hbm.at[pl.ds(w * CAP, CAP)])
    pltpu.sync_copy(lg1_v, lg1_hbm.at[pl.ds(w * CAP, CAP)])
    pltpu.sync_copy(ls1_v, ls1_hbm.at[pl.ds(w * CAP, CAP)])
    pltpu.sync_copy(cnt_v, cnts_hbm.at[pl.ds(w * 16, 16)])


@functools.partial(
    pl.kernel,
    mesh=plsc.VectorSubcoreMesh(core_axis_name="c", subcore_axis_name="s"),
    compiler_params=pltpu.CompilerParams(needs_layout_passes=False),
    out_type=[
        jax.ShapeDtypeStruct((NW * CAP,), jnp.int32),
        jax.ShapeDtypeStruct((NW * CAP,), jnp.int32),
        jax.ShapeDtypeStruct((NW * CAP,), jnp.int32),
        jax.ShapeDtypeStruct((NW * CAP,), jnp.int32),
        jax.ShapeDtypeStruct((NW * 16,), jnp.int32),
    ],
    scratch_types=[
        pltpu.VMEM((PW // 128, 128), jnp.int32),
        pltpu.VMEM((PW // 128, 128), jnp.int32),
        pltpu.VMEM((CAP,), jnp.int32),
        pltpu.VMEM((CAP,), jnp.int32),
        pltpu.VMEM((CAP,), jnp.int32),
        pltpu.VMEM((CAP,), jnp.int32),
        pltpu.VMEM((16,), jnp.int32),
    ],
)
def _sc_prepass(*refs):
    _sc_prepass_body(*refs)


def _sc_scatter_body(table_hbm, lg0_hbm, lg1_hbm, ls0_hbm, ls1_hbm, cnts_hbm,
                     zeros_hbm, out_hbm,
                     lg_v, ls_v, cnt_v, rows_a, rows_b, rows_c, rows_d,
                     acc_sh, sem_a, sem_b, sem_c, sem_d):
    c = lax.axis_index("c")
    s = lax.axis_index("s")

    @pl.when(s < NS - 1)
    def _():
        pltpu.sync_copy(zeros_hbm.at[pl.ds(s * RPT, RPT)],
                        acc_sh.at[pl.ds(s * RPT, RPT)])

    @pl.when(s == NS - 1)
    def _():
        pltpu.sync_copy(zeros_hbm.at[pl.ds((NS - 1) * RPT, RPT_LAST)],
                        acc_sh.at[pl.ds((NS - 1) * RPT, RPT_LAST)])

    plsc.subcore_barrier()

    bufs = [(rows_a, sem_a), (rows_b, sem_b), (rows_c, sem_c), (rows_d, sem_d)]

    def gather(k, buf, sem):
        pltpu.async_copy(table_hbm.at[lg_v.at[pl.ds(k * CHG, CHG)]], buf, sem)

    def drain_scatter(k, buf, sem):
        pltpu.make_async_copy(
            table_hbm.at[lg_v.at[pl.ds(k * CHG, CHG)]], buf, sem).wait()
        pltpu.sync_copy(
            buf, acc_sh.at[ls_v.at[k // CPR, pl.ds((k % CPR) * CHG, CHG)]],
            add=True)

    for li in range(2):
        w = 2 * s + li

        @pl.when(c == 0)
        def _():
            pltpu.sync_copy(lg0_hbm.at[pl.ds(w * CAP, CAP)], lg_v)
            pltpu.sync_copy(ls0_hbm.at[w], ls_v)

        @pl.when(c == 1)
        def _():
            pltpu.sync_copy(lg1_hbm.at[pl.ds(w * CAP, CAP)], lg_v)
            pltpu.sync_copy(ls1_hbm.at[w], ls_v)

        pltpu.sync_copy(cnts_hbm.at[pl.ds(w * 16, 16)], cnt_v)
        io = lax.iota(jnp.int32, 16)
        cnt = jnp.sum(jnp.where(io == c, cnt_v[...], 0))
        nch = cnt // CHG
        n4 = cnt // PAD_CH

        for q in range(NBUF - 1):
            @pl.when(q < nch)
            def _():
                gather(q, *bufs[q])

        def chunk(j, carry):
            for q in range(NBUF):
                k = NBUF * j + q
                kn = k + NBUF - 1

                @pl.when(kn < nch)
                def _():
                    gather(kn, *bufs[(q + NBUF - 1) % NBUF])

                drain_scatter(k, *bufs[q])
            return carry

        lax.fori_loop(0, n4, chunk, 0, unroll=False)

    plsc.subcore_barrier()

    @pl.when(s < NS - 1)
    def _():
        pltpu.sync_copy(acc_sh.at[pl.ds(s * RPT, RPT)],
                        out_hbm.at[c, pl.ds(s * RPT, RPT)])

    @pl.when(s == NS - 1)
    def _():
        pltpu.sync_copy(acc_sh.at[pl.ds((NS - 1) * RPT, RPT_LAST)],
                        out_hbm.at[c, pl.ds((NS - 1) * RPT, RPT_LAST)])


@functools.partial(
    pl.kernel,
    mesh=plsc.VectorSubcoreMesh(core_axis_name="c", subcore_axis_name="s"),
    compiler_params=pltpu.CompilerParams(needs_layout_passes=False),
    out_type=jax.ShapeDtypeStruct((NC, NRA, D), jnp.float32),
    scratch_types=[
        pltpu.VMEM((CAP,), jnp.int32),
        pltpu.VMEM((CAP // 128, 128), jnp.int32),
        pltpu.VMEM((16,), jnp.int32),
        pltpu.VMEM((CHG, D), jnp.float32),
        pltpu.VMEM((CHG, D), jnp.float32),
        pltpu.VMEM((CHG, D), jnp.float32),
        pltpu.VMEM((CHG, D), jnp.float32),
        pltpu.VMEM_SHARED((NRA, D), jnp.float32),
        pltpu.SemaphoreType.DMA,
        pltpu.SemaphoreType.DMA,
        pltpu.SemaphoreType.DMA,
        pltpu.SemaphoreType.DMA,
    ],
)
def _sc_scatter(*refs):
    _sc_scatter_body(*refs)


def _lin2_kernel(x_ref, w0_ref, b0_ref, w1_ref, b1_ref, o0_ref, o1_ref):
    x = x_ref[...]
    o0_ref[...] = jnp.dot(x, w0_ref[...], preferred_element_type=jnp.float32) + b0_ref[...]
    o1_ref[...] = jnp.dot(x, w1_ref[...], preferred_element_type=jnp.float32) + b1_ref[...]


def _lin2_fused_kernel(vw0_ref, ns_ref, w0_ref, b0_ref, w1_ref, b1_ref,
                       o0_ref, o1_ref):
    h = jax.nn.relu(vw0_ref[...] + ns_ref[0])
    o0_ref[...] = jnp.dot(h, w0_ref[...], preferred_element_type=jnp.float32) + b0_ref[...]
    o1_ref[...] = jnp.dot(h, w1_ref[...], preferred_element_type=jnp.float32) + b1_ref[...]


def _final_kernel(x_ref, vw0_ref, ns_ref, o_ref):
    h = jax.nn.relu(vw0_ref[...] + ns_ref[0])
    o_ref[...] = (x_ref[...] + h) * 0.5


_W_SPEC = pl.BlockSpec((D, D), lambda i: (0, 0))
_B_SPEC = pl.BlockSpec((1, D), lambda i: (0, 0))
_X_SPEC = pl.BlockSpec((BM, D), lambda i: (i, 0))
_NS_SPEC = pl.BlockSpec((1, BM, D), lambda i: (i // 5, i % 5, 0))
_OUT2_SHAPE = [jax.ShapeDtypeStruct((N, D), jnp.float32)] * 2


def _lin2(x, w0t, b0, w1t, b1):
    return pl.pallas_call(
        _lin2_kernel,
        grid=(N // BM,),
        in_specs=[_X_SPEC, _W_SPEC, _B_SPEC, _W_SPEC, _B_SPEC],
        out_specs=[_X_SPEC, _X_SPEC],
        out_shape=_OUT2_SHAPE,
    )(x, w0t, b0.reshape(1, D), w1t, b1.reshape(1, D))


def _lin2_fused(vw0, ns, w0t, b0, w1t, b1):
    return pl.pallas_call(
        _lin2_fused_kernel,
        grid=(N // BM,),
        in_specs=[_X_SPEC, _NS_SPEC, _W_SPEC, _B_SPEC, _W_SPEC, _B_SPEC],
        out_specs=[_X_SPEC, _X_SPEC],
        out_shape=_OUT2_SHAPE,
    )(vw0, ns, w0t, b0.reshape(1, D), w1t, b1.reshape(1, D))


def _final(x, vw0, ns):
    return pl.pallas_call(
        _final_kernel,
        grid=(N // BM,),
        in_specs=[_X_SPEC, _X_SPEC, _NS_SPEC],
        out_specs=_X_SPEC,
        out_shape=jax.ShapeDtypeStruct((N, D), jnp.float32),
    )(x, vw0, ns)


def kernel(input_feats, edges_packed, W0_1, b0_1, W1_1, b1_1,
           W0_2, b0_2, W1_2, b1_2):
    src = edges_packed[:, 0].astype(jnp.int32)
    dst = edges_packed[:, 1].astype(jnp.int32)
    padz = jnp.zeros((PPAD - PAIRS,), jnp.int32)
    gsrc = jnp.concatenate([dst, src, padz]).reshape(NW, PW // 128, 128)
    tgt = jnp.concatenate([src, dst, padz - 1]).reshape(NW, PW // 128, 128)

    lg0, ls0, lg1, ls1, cnts = _sc_prepass(gsrc, tgt)
    ls0 = ls0.reshape(NW, CAP // 128, 128)
    ls1 = ls1.reshape(NW, CAP // 128, 128)
    zeros = jnp.zeros((NRA, D), jnp.float32)

    vw0_1, vw1_1 = _lin2(input_feats, W0_1.T, b0_1, W1_1.T, b1_1)
    ns1 = _sc_scatter(vw1_1, lg0, lg1, ls0, ls1, cnts, zeros)
    vw0_2, vw1_2 = _lin2_fused(vw0_1, ns1, W0_2.T, b0_2, W1_2.T, b1_2)
    ns2 = _sc_scatter(vw1_2, lg0, lg1, ls0, ls1, cnts, zeros)
    return _final(input_feats, vw0_2, ns2)
